# baseline (speedup 1.0000x reference)
import functools

import numpy as np
import jax
import jax.numpy as jnp
from jax import lax
from jax.experimental import pallas as pl
from jax.experimental.pallas import tpu as pltpu
from jax.experimental.pallas import tpu_sc as plsc

_N = 10000
_D = 8
_DE = 4
_NC = 2
_NS = 16
_NW = _NC * _NS
_CHUNK = 128
_LAG = 8
_T = 4096


def _sc_mesh():
    return plsc.VectorSubcoreMesh(core_axis_name="c", subcore_axis_name="s")


def _make_gather(e_pad, nchunk):
    epw = nchunk * _CHUNK

    @functools.partial(
        pl.kernel,
        mesh=_sc_mesh(),
        out_type=jax.ShapeDtypeStruct((e_pad // _CHUNK, _CHUNK, _D),
                                      jnp.float32),
        scratch_types=[
            pltpu.VMEM((nchunk, _CHUNK), jnp.int32),
            pltpu.VMEM((nchunk, _CHUNK, _D), jnp.float32),
            pltpu.SemaphoreType.DMA,
        ],
        compiler_params=pltpu.CompilerParams(use_tc_tiling_on_sc=False),
    )
    def gather(table_hbm, idx_hbm, out_hbm, idx_v, rows_v, sem):
        wid = lax.axis_index("s") * _NC + lax.axis_index("c")
        pltpu.sync_copy(idx_hbm.at[wid], idx_v)

        def fire(j):
            pltpu.async_copy(
                table_hbm.at[idx_v.at[j]],
                rows_v.at[j],
                sem,
            )

        for j in range(_LAG):
            fire(j)

        def step(j, carry):
            @pl.when(j + _LAG < nchunk)
            def _():
                fire(j + _LAG)

            pltpu.make_async_copy(
                table_hbm.at[idx_v.at[j]],
                rows_v.at[j],
                sem,
            ).wait()
            return carry

        lax.fori_loop(0, nchunk, step, 0)
        pltpu.sync_copy(rows_v, out_hbm.at[pl.ds(wid * nchunk, nchunk)])

    return gather


def _make_scatter(e_pad, nchunk, with_counts):
    epw = nchunk * _CHUNK
    nacc = _N + _NS
    stripe_i = nacc // _NS
    stripe_o = _N // _NS

    out_type = [jax.ShapeDtypeStruct((_NC, _N, _D), jnp.float32)]
    scratch = [
        pltpu.VMEM((nchunk, _CHUNK), jnp.int32),
        pltpu.VMEM((nchunk, _CHUNK, _D), jnp.float32),
        pltpu.VMEM_SHARED((nacc, _D), jnp.float32),
    ]
    if with_counts:
        out_type.append(jax.ShapeDtypeStruct((_NC, _N, _D), jnp.float32))
        scratch.append(pltpu.VMEM((_CHUNK, _D), jnp.float32))
        scratch.append(pltpu.VMEM_SHARED((nacc, _D), jnp.float32))
    scratch.append(pltpu.SemaphoreType.DMA)
    if with_counts:
        scratch.append(pltpu.SemaphoreType.DMA)

    def body(*refs):
        if with_counts:
            (msg_hbm, idx_hbm, zeros_hbm, ones_hbm, sums_hbm, cnt_hbm,
             idx_v, msg_v, acc_sh, ones_v, cnt_sh, sem, sem2) = refs
        else:
            (msg_hbm, idx_hbm, zeros_hbm, sums_hbm,
             idx_v, msg_v, acc_sh, sem) = refs
        c = lax.axis_index("c")
        s = lax.axis_index("s")
        wid = s * _NC + c
        pltpu.sync_copy(idx_hbm.at[wid], idx_v)
        pltpu.sync_copy(msg_hbm.at[pl.ds(wid * nchunk, nchunk)], msg_v)
        pltpu.sync_copy(
            zeros_hbm.at[pl.ds(s * stripe_i, stripe_i)],
            acc_sh.at[pl.ds(s * stripe_i, stripe_i)],
        )
        if with_counts:
            pltpu.sync_copy(ones_hbm, ones_v)
            pltpu.sync_copy(
                zeros_hbm.at[pl.ds(s * stripe_i, stripe_i)],
                cnt_sh.at[pl.ds(s * stripe_i, stripe_i)],
            )
        plsc.subcore_barrier()

        def fire(j):
            pltpu.async_copy(
                msg_v.at[j],
                acc_sh.at[idx_v.at[j]],
                sem,
                add=True,
            )
            if with_counts:
                pltpu.async_copy(ones_v, cnt_sh.at[idx_v.at[j]], sem2,
                                 add=True)

        for j in range(_LAG):
            fire(j)

        def step(j, carry):
            @pl.when(j + _LAG < nchunk)
            def _():
                fire(j + _LAG)

            pltpu.make_async_copy(
                msg_v.at[j],
                acc_sh.at[idx_v.at[j]],
                sem,
            ).wait()
            if with_counts:
                pltpu.make_async_copy(
                    ones_v, cnt_sh.at[idx_v.at[j]], sem2).wait()
            return carry

        lax.fori_loop(0, nchunk, step, 0)
        plsc.subcore_barrier()
        pltpu.sync_copy(
            acc_sh.at[pl.ds(s * stripe_o, stripe_o)],
            sums_hbm.at[c, pl.ds(s * stripe_o, stripe_o)],
        )
        if with_counts:
            pltpu.sync_copy(
                cnt_sh.at[pl.ds(s * stripe_o, stripe_o)],
                cnt_hbm.at[c, pl.ds(s * stripe_o, stripe_o)],
            )

    return functools.partial(
        pl.kernel,
        mesh=_sc_mesh(),
        out_type=tuple(out_type) if with_counts else out_type[0],
        scratch_types=scratch,
        compiler_params=pltpu.CompilerParams(use_tc_tiling_on_sc=False),
    )(body)


def _mlp_body(ea, xs, W1, b1, W2, b2, rep, ssum, out):
    nb = _T // _CHUNK
    h = jnp.maximum(
        jnp.dot(ea[...], W1[...], preferred_element_type=jnp.float32) + b1[...],
        0.0,
    )
    w = jnp.dot(h, W2[...], preferred_element_type=jnp.float32) + b2[...]
    xs2 = xs[...].reshape(_T, _D)
    xr = jnp.dot(xs2, rep[...], preferred_element_type=jnp.float32)
    msg = jnp.dot(xr * w, ssum[...], preferred_element_type=jnp.float32)
    out[...] = msg.reshape(nb, _CHUNK, _D)


def _mlp_call(ea, xs, W1, b1, W2, b2, rep, ssum):
    e_pad = ea.shape[0]
    nb = _T // _CHUNK
    dd = _D * _D
    grid = e_pad // _T
    zero = lambda i: (0, 0)
    return pl.pallas_call(
        _mlp_body,
        grid=(grid,),
        in_specs=[
            pl.BlockSpec((_T, _DE), lambda i: (i, 0)),
            pl.BlockSpec((nb, _CHUNK, _D), lambda i: (i, 0, 0)),
            pl.BlockSpec((_DE, dd), zero),
            pl.BlockSpec((1, dd), zero),
            pl.BlockSpec((dd, dd), zero),
            pl.BlockSpec((1, dd), zero),
            pl.BlockSpec((_D, dd), zero),
            pl.BlockSpec((dd, _D), zero),
        ],
        out_specs=pl.BlockSpec((nb, _CHUNK, _D), lambda i: (i, 0, 0)),
        out_shape=jax.ShapeDtypeStruct((e_pad // _CHUNK, _CHUNK, _D),
                                       jnp.float32),
    )(ea, xs, W1, b1, W2, b2, rep, ssum)


def _combine_body(sums, cnt, x, R, bias, out, *, relu):
    tot = sums[0] + sums[1]
    c = cnt[0, :, 0:1] + cnt[1, :, 0:1]
    agg = tot / jnp.maximum(c, 1.0)
    o = agg + jnp.dot(x[...], R[...], preferred_element_type=jnp.float32) + bias[...]
    out[...] = jnp.maximum(o, 0.0) if relu else o


def _combine_call(sums, cnt, x, R, bias, relu):
    return pl.pallas_call(
        functools.partial(_combine_body, relu=relu),
        out_shape=jax.ShapeDtypeStruct((_N, _D), jnp.float32),
    )(sums, cnt, x, R, bias)


def kernel(x, edge_index, edge_attr, W1_0, b1_0, W2_0, b2_0, R_0, bias_0,
           W1_1, b1_1, W2_1, b2_1, R_1, bias_1):
    e = edge_attr.shape[0]
    nchunk = -(-e // (_NW * _CHUNK))
    e_pad = _NW * _CHUNK * nchunk
    pad = e_pad - e

    src = jnp.concatenate(
        [edge_index[0], jnp.zeros((pad,), jnp.int32)]).reshape(_NW, nchunk, _CHUNK)
    dst = jnp.concatenate(
        [edge_index[1], jnp.full((pad,), _N, jnp.int32)]).reshape(_NW, nchunk, _CHUNK)
    ea = jnp.concatenate(
        [edge_attr, jnp.zeros((pad, _DE), jnp.float32)], axis=0)

    zeros = jnp.zeros((_N + _NS, _D), jnp.float32)
    ones = jnp.ones((_CHUNK, _D), jnp.float32)
    rep = jnp.asarray(np.kron(np.eye(_D), np.ones((1, _D))), jnp.float32)
    ssum = jnp.asarray(np.kron(np.ones((_D, 1)), np.eye(_D)), jnp.float32)

    gather = _make_gather(e_pad, nchunk)
    scatter0 = _make_scatter(e_pad, nchunk, True)
    scatter1 = _make_scatter(e_pad, nchunk, False)

    layers = [
        (W1_0, b1_0, W2_0, b2_0, R_0, bias_0, True),
        (W1_1, b1_1, W2_1, b2_1, R_1, bias_1, False),
    ]
    h = x
    cnt = None
    for W1, b1, W2, b2, R, bias, relu in layers:
        xs = gather(h, src)
        msg = _mlp_call(ea, xs, W1, b1.reshape(1, -1), W2, b2.reshape(1, -1),
                        rep, ssum)
        if cnt is None:
            sums, cnt = scatter0(msg, dst, zeros, ones)
        else:
            sums = scatter1(msg, dst, zeros)
        h = _combine_call(sums, cnt, h, R, bias.reshape(1, -1), relu)
    return h


# speedup vs baseline: 1.2637x; 1.2637x over previous
import functools

import numpy as np
import jax
import jax.numpy as jnp
from jax import lax
from jax.experimental import pallas as pl
from jax.experimental.pallas import tpu as pltpu
from jax.experimental.pallas import tpu_sc as plsc

_N = 10000
_D = 8
_DE = 4
_NC = 2
_NS = 16
_NW = _NC * _NS
_CHUNK = 128
_LAG = 8
_T = 4096


def _sc_mesh():
    return plsc.VectorSubcoreMesh(core_axis_name="c", subcore_axis_name="s")


def _make_gather(e_pad, nchunk):
    epw = nchunk * _CHUNK

    @functools.partial(
        pl.kernel,
        mesh=_sc_mesh(),
        out_type=jax.ShapeDtypeStruct((e_pad // _CHUNK, _CHUNK, _D),
                                      jnp.float32),
        scratch_types=[
            pltpu.VMEM((nchunk, _CHUNK), jnp.int32),
            pltpu.VMEM((nchunk, _CHUNK, _D), jnp.float32),
            pltpu.SemaphoreType.DMA,
        ],
        compiler_params=pltpu.CompilerParams(use_tc_tiling_on_sc=False),
    )
    def gather(table_hbm, idx_hbm, out_hbm, idx_v, rows_v, sem):
        wid = lax.axis_index("s") * _NC + lax.axis_index("c")
        pltpu.sync_copy(idx_hbm.at[wid], idx_v)

        def fire(j):
            pltpu.async_copy(
                table_hbm.at[idx_v.at[j]],
                rows_v.at[j],
                sem,
            )

        for j in range(_LAG):
            fire(j)

        def step(j, carry):
            @pl.when(j + _LAG < nchunk)
            def _():
                fire(j + _LAG)

            pltpu.make_async_copy(
                table_hbm.at[idx_v.at[j]],
                rows_v.at[j],
                sem,
            ).wait()
            return carry

        lax.fori_loop(0, nchunk, step, 0)
        pltpu.sync_copy(rows_v, out_hbm.at[pl.ds(wid * nchunk, nchunk)])

    return gather


def _make_scatter(e_pad, nchunk, with_counts):
    epw = nchunk * _CHUNK
    nacc = _N + _NS
    stripe_i = nacc // _NS
    stripe_o = _N // _NS

    out_type = [jax.ShapeDtypeStruct((_NC, _N, _D), jnp.float32)]
    scratch = [
        pltpu.VMEM((nchunk, _CHUNK), jnp.int32),
        pltpu.VMEM((nchunk, _CHUNK, _D), jnp.float32),
        pltpu.VMEM_SHARED((nacc, _D), jnp.float32),
    ]
    if with_counts:
        out_type.append(jax.ShapeDtypeStruct((_NC, _N, _D), jnp.float32))
        scratch.append(pltpu.VMEM((_CHUNK, _D), jnp.float32))
        scratch.append(pltpu.VMEM_SHARED((nacc, _D), jnp.float32))
    scratch.append(pltpu.SemaphoreType.DMA)
    if with_counts:
        scratch.append(pltpu.SemaphoreType.DMA)

    def body(*refs):
        if with_counts:
            (msg_hbm, idx_hbm, zeros_hbm, ones_hbm, sums_hbm, cnt_hbm,
             idx_v, msg_v, acc_sh, ones_v, cnt_sh, sem, sem2) = refs
        else:
            (msg_hbm, idx_hbm, zeros_hbm, sums_hbm,
             idx_v, msg_v, acc_sh, sem) = refs
        c = lax.axis_index("c")
        s = lax.axis_index("s")
        wid = s * _NC + c
        pltpu.sync_copy(idx_hbm.at[wid], idx_v)
        pltpu.sync_copy(msg_hbm.at[pl.ds(wid * nchunk, nchunk)], msg_v)
        pltpu.sync_copy(
            zeros_hbm.at[pl.ds(s * stripe_i, stripe_i)],
            acc_sh.at[pl.ds(s * stripe_i, stripe_i)],
        )
        if with_counts:
            pltpu.sync_copy(ones_hbm, ones_v)
            pltpu.sync_copy(
                zeros_hbm.at[pl.ds(s * stripe_i, stripe_i)],
                cnt_sh.at[pl.ds(s * stripe_i, stripe_i)],
            )
        plsc.subcore_barrier()

        def fire(j):
            pltpu.async_copy(
                msg_v.at[j],
                acc_sh.at[idx_v.at[j]],
                sem,
                add=True,
            )
            if with_counts:
                pltpu.async_copy(ones_v, cnt_sh.at[idx_v.at[j]], sem2,
                                 add=True)

        for j in range(_LAG):
            fire(j)

        def step(j, carry):
            @pl.when(j + _LAG < nchunk)
            def _():
                fire(j + _LAG)

            pltpu.make_async_copy(
                msg_v.at[j],
                acc_sh.at[idx_v.at[j]],
                sem,
            ).wait()
            if with_counts:
                pltpu.make_async_copy(
                    ones_v, cnt_sh.at[idx_v.at[j]], sem2).wait()
            return carry

        lax.fori_loop(0, nchunk, step, 0)
        plsc.subcore_barrier()
        pltpu.sync_copy(
            acc_sh.at[pl.ds(s * stripe_o, stripe_o)],
            sums_hbm.at[c, pl.ds(s * stripe_o, stripe_o)],
        )
        if with_counts:
            pltpu.sync_copy(
                cnt_sh.at[pl.ds(s * stripe_o, stripe_o)],
                cnt_hbm.at[c, pl.ds(s * stripe_o, stripe_o)],
            )

    return functools.partial(
        pl.kernel,
        mesh=_sc_mesh(),
        out_type=tuple(out_type) if with_counts else out_type[0],
        scratch_types=scratch,
        compiler_params=pltpu.CompilerParams(use_tc_tiling_on_sc=False),
    )(body)


def _mlp_body(ea, xs, W1, b1, W2, b2, rep, ssum, out):
    eap = ea[...]
    xsp = xs[...]
    W1v = W1[...]
    b1v = b1[...]
    W2v = W2[...]
    b2v = b2[...]
    repv = rep[...]
    ssumv = ssum[...]
    pieces = []
    for k in range(16):
        ea_k = eap[:, k * _D:(k + 1) * _D]
        xs_k = xsp[:, k * _D:(k + 1) * _D]
        h_k = jnp.maximum(
            jnp.dot(ea_k, W1v, preferred_element_type=jnp.float32) + b1v, 0.0)
        w_k = jnp.dot(h_k, W2v, preferred_element_type=jnp.float32) + b2v
        xr_k = jnp.dot(xs_k, repv, preferred_element_type=jnp.float32)
        pieces.append(
            jnp.dot(xr_k * w_k, ssumv, preferred_element_type=jnp.float32))
    out[...] = jnp.concatenate(pieces, axis=1)


def _mlp_call(ea, xs, W1, b1, W2, b2, rep, ssum):
    rows = ea.shape[0]
    tb = _T // 16
    dd = _D * _D
    grid = rows // tb
    zero = lambda i: (0, 0)
    return pl.pallas_call(
        _mlp_body,
        grid=(grid,),
        in_specs=[
            pl.BlockSpec((tb, 128), lambda i: (i, 0)),
            pl.BlockSpec((tb, 128), lambda i: (i, 0)),
            pl.BlockSpec((_D, dd), zero),
            pl.BlockSpec((1, dd), zero),
            pl.BlockSpec((dd, dd), zero),
            pl.BlockSpec((1, dd), zero),
            pl.BlockSpec((_D, dd), zero),
            pl.BlockSpec((dd, _D), zero),
        ],
        out_specs=pl.BlockSpec((tb, 128), lambda i: (i, 0)),
        out_shape=jax.ShapeDtypeStruct((rows, 128), jnp.float32),
    )(ea, xs, W1, b1, W2, b2, rep, ssum)


def _combine_body(sums, cnt, x, R, bias, out, *, relu):
    tot = sums[0] + sums[1]
    c = cnt[0, :, 0:1] + cnt[1, :, 0:1]
    agg = tot / jnp.maximum(c, 1.0)
    o = agg + jnp.dot(x[...], R[...], preferred_element_type=jnp.float32) + bias[...]
    out[...] = jnp.maximum(o, 0.0) if relu else o


def _combine_call(sums, cnt, x, R, bias, relu):
    return pl.pallas_call(
        functools.partial(_combine_body, relu=relu),
        out_shape=jax.ShapeDtypeStruct((_N, _D), jnp.float32),
    )(sums, cnt, x, R, bias)


def kernel(x, edge_index, edge_attr, W1_0, b1_0, W2_0, b2_0, R_0, bias_0,
           W1_1, b1_1, W2_1, b2_1, R_1, bias_1):
    e = edge_attr.shape[0]
    nchunk = -(-e // (_NW * _CHUNK))
    e_pad = _NW * _CHUNK * nchunk
    pad = e_pad - e

    src = jnp.concatenate(
        [edge_index[0], jnp.zeros((pad,), jnp.int32)]).reshape(_NW, nchunk, _CHUNK)
    dst = jnp.concatenate(
        [edge_index[1], jnp.full((pad,), _N, jnp.int32)]).reshape(_NW, nchunk, _CHUNK)
    ea = jnp.pad(edge_attr, ((0, pad), (0, _D - _DE))).reshape(e_pad // 16, 128)

    zeros = jnp.zeros((_N + _NS, _D), jnp.float32)
    ones = jnp.ones((_CHUNK, _D), jnp.float32)
    rep = jnp.asarray(np.kron(np.eye(_D), np.ones((1, _D))), jnp.float32)
    ssum = jnp.asarray(np.kron(np.ones((_D, 1)), np.eye(_D)), jnp.float32)

    gather = _make_gather(e_pad, nchunk)
    scatter0 = _make_scatter(e_pad, nchunk, True)
    scatter1 = _make_scatter(e_pad, nchunk, False)

    layers = [
        (W1_0, b1_0, W2_0, b2_0, R_0, bias_0, True),
        (W1_1, b1_1, W2_1, b2_1, R_1, bias_1, False),
    ]
    h = x
    cnt = None
    for W1, b1, W2, b2, R, bias, relu in layers:
        xs = gather(h, src).reshape(e_pad // 16, 128)
        W1p = jnp.pad(W1, ((0, _D - _DE), (0, 0)))
        msg = _mlp_call(ea, xs, W1p, b1.reshape(1, -1), W2, b2.reshape(1, -1),
                        rep, ssum)
        msg = msg.reshape(e_pad // _CHUNK, _CHUNK, _D)
        if cnt is None:
            sums, cnt = scatter0(msg, dst, zeros, ones)
        else:
            sums = scatter1(msg, dst, zeros)
        h = _combine_call(sums, cnt, h, R, bias.reshape(1, -1), relu)
    return h


# speedup vs baseline: 1.7063x; 1.3503x over previous
import functools

import numpy as np
import jax
import jax.numpy as jnp
from jax import lax
from jax.experimental import pallas as pl
from jax.experimental.pallas import tpu as pltpu
from jax.experimental.pallas import tpu_sc as plsc

_N = 10000
_D = 8
_DE = 4
_NC = 2
_NS = 16
_NW = _NC * _NS
_CHUNK = 128
_LAG = 8
_T = 4096


def _sc_mesh():
    return plsc.VectorSubcoreMesh(core_axis_name="c", subcore_axis_name="s")


def _make_gather(e_pad, nchunk):
    epw = nchunk * _CHUNK

    @functools.partial(
        pl.kernel,
        mesh=_sc_mesh(),
        out_type=jax.ShapeDtypeStruct((e_pad // _CHUNK, _CHUNK, _D),
                                      jnp.float32),
        scratch_types=[
            pltpu.VMEM((nchunk, _CHUNK), jnp.int32),
            pltpu.VMEM((nchunk, _CHUNK, _D), jnp.float32),
            pltpu.SemaphoreType.DMA,
        ],
        compiler_params=pltpu.CompilerParams(use_tc_tiling_on_sc=False),
    )
    def gather(table_hbm, idx_hbm, out_hbm, idx_v, rows_v, sem):
        wid = lax.axis_index("s") * _NC + lax.axis_index("c")
        pltpu.sync_copy(idx_hbm.at[wid], idx_v)

        def fire(j):
            pltpu.async_copy(
                table_hbm.at[idx_v.at[j]],
                rows_v.at[j],
                sem,
            )

        for j in range(_LAG):
            fire(j)

        def step(j, carry):
            @pl.when(j + _LAG < nchunk)
            def _():
                fire(j + _LAG)

            pltpu.make_async_copy(
                table_hbm.at[idx_v.at[j]],
                rows_v.at[j],
                sem,
            ).wait()
            return carry

        lax.fori_loop(0, nchunk, step, 0)
        pltpu.sync_copy(rows_v, out_hbm.at[pl.ds(wid * nchunk, nchunk)])

    return gather


def _make_eapack(e_pad, nchunk):
    nblk = e_pad // _CHUNK

    @functools.partial(
        pl.kernel,
        mesh=_sc_mesh(),
        out_type=jax.ShapeDtypeStruct((nblk, _CHUNK * _D), jnp.float32),
        scratch_types=[
            pltpu.VMEM((nchunk, _DE, _CHUNK), jnp.float32),
            pltpu.VMEM((nchunk, _CHUNK * _D), jnp.float32),
        ],
        compiler_params=pltpu.CompilerParams(use_tc_tiling_on_sc=False,
                                             needs_layout_passes=False),
    )
    def eapack(ea3_hbm, out_hbm, strip_v, rows_v):
        wid = lax.axis_index("s") * _NC + lax.axis_index("c")
        pltpu.sync_copy(ea3_hbm.at[pl.ds(wid * nchunk, nchunk)], strip_v)
        zeros16 = jnp.zeros((16,), jnp.float32)
        lane = lax.iota(jnp.int32, 16) * _D

        def step(j, carry):
            jv = jnp.zeros((16,), jnp.int32) + j
            for g in range(_CHUNK // 16):
                pos0 = lane + (g * 16 * _D)
                for f in range(_D):
                    if f < _DE:
                        val = strip_v[j, f, pl.ds(g * 16, 16)]
                    else:
                        val = zeros16
                    plsc.store_scatter(rows_v, [jv, pos0 + f], val)
            return carry

        lax.fori_loop(0, nchunk, step, 0)
        pltpu.sync_copy(rows_v, out_hbm.at[pl.ds(wid * nchunk, nchunk)])

    return eapack


def _make_scatter(e_pad, nchunk, with_counts):
    epw = nchunk * _CHUNK
    nacc = _N + _NS
    stripe_i = nacc // _NS
    stripe_o = _N // _NS

    out_type = [jax.ShapeDtypeStruct((_NC, _N, _D), jnp.float32)]
    scratch = [
        pltpu.VMEM((nchunk, _CHUNK), jnp.int32),
        pltpu.VMEM((nchunk, _CHUNK, _D), jnp.float32),
        pltpu.VMEM_SHARED((nacc, _D), jnp.float32),
    ]
    if with_counts:
        out_type.append(jax.ShapeDtypeStruct((_NC, _N, _D), jnp.float32))
        scratch.append(pltpu.VMEM((_CHUNK, _D), jnp.float32))
        scratch.append(pltpu.VMEM_SHARED((nacc, _D), jnp.float32))
    scratch.append(pltpu.SemaphoreType.DMA)
    if with_counts:
        scratch.append(pltpu.SemaphoreType.DMA)

    def body(*refs):
        if with_counts:
            (msg_hbm, idx_hbm, zeros_hbm, ones_hbm, sums_hbm, cnt_hbm,
             idx_v, msg_v, acc_sh, ones_v, cnt_sh, sem, sem2) = refs
        else:
            (msg_hbm, idx_hbm, zeros_hbm, sums_hbm,
             idx_v, msg_v, acc_sh, sem) = refs
        c = lax.axis_index("c")
        s = lax.axis_index("s")
        wid = s * _NC + c
        pltpu.sync_copy(idx_hbm.at[wid], idx_v)
        pltpu.sync_copy(msg_hbm.at[pl.ds(wid * nchunk, nchunk)], msg_v)
        pltpu.sync_copy(
            zeros_hbm.at[pl.ds(s * stripe_i, stripe_i)],
            acc_sh.at[pl.ds(s * stripe_i, stripe_i)],
        )
        if with_counts:
            pltpu.sync_copy(ones_hbm, ones_v)
            pltpu.sync_copy(
                zeros_hbm.at[pl.ds(s * stripe_i, stripe_i)],
                cnt_sh.at[pl.ds(s * stripe_i, stripe_i)],
            )
        plsc.subcore_barrier()

        def fire(j):
            pltpu.async_copy(
                msg_v.at[j],
                acc_sh.at[idx_v.at[j]],
                sem,
                add=True,
            )
            if with_counts:
                pltpu.async_copy(ones_v, cnt_sh.at[idx_v.at[j]], sem2,
                                 add=True)

        for j in range(_LAG):
            fire(j)

        def step(j, carry):
            @pl.when(j + _LAG < nchunk)
            def _():
                fire(j + _LAG)

            pltpu.make_async_copy(
                msg_v.at[j],
                acc_sh.at[idx_v.at[j]],
                sem,
            ).wait()
            if with_counts:
                pltpu.make_async_copy(
                    ones_v, cnt_sh.at[idx_v.at[j]], sem2).wait()
            return carry

        lax.fori_loop(0, nchunk, step, 0)
        plsc.subcore_barrier()
        pltpu.sync_copy(
            acc_sh.at[pl.ds(s * stripe_o, stripe_o)],
            sums_hbm.at[c, pl.ds(s * stripe_o, stripe_o)],
        )
        if with_counts:
            pltpu.sync_copy(
                cnt_sh.at[pl.ds(s * stripe_o, stripe_o)],
                cnt_hbm.at[c, pl.ds(s * stripe_o, stripe_o)],
            )

    return functools.partial(
        pl.kernel,
        mesh=_sc_mesh(),
        out_type=tuple(out_type) if with_counts else out_type[0],
        scratch_types=scratch,
        compiler_params=pltpu.CompilerParams(use_tc_tiling_on_sc=False),
    )(body)


def _mlp_body(ea, xs, W1, b1, W2, b2, rep, ssum, out):
    eap = ea[...]
    xsp = xs[...]
    W1v = W1[...]
    b1v = b1[...]
    W2v = W2[...]
    b2v = b2[...]
    repv = rep[...]
    ssumv = ssum[...]
    W2b = W2v.astype(jnp.bfloat16)
    pieces = []
    for k in range(16):
        ea_k = eap[:, k * _D:(k + 1) * _D]
        xs_k = xsp[:, k * _D:(k + 1) * _D]
        h_k = jnp.maximum(
            jnp.dot(ea_k, W1v, preferred_element_type=jnp.float32) + b1v, 0.0)
        w_k = jnp.dot(h_k.astype(jnp.bfloat16), W2b,
                      preferred_element_type=jnp.float32) + b2v
        xr_k = jnp.dot(xs_k, repv, preferred_element_type=jnp.float32)
        pieces.append(
            jnp.dot(xr_k * w_k, ssumv, preferred_element_type=jnp.float32))
    out[...] = jnp.concatenate(pieces, axis=1)


def _mlp_call(ea, xs, W1, b1, W2, b2, rep, ssum):
    rows = ea.shape[0]
    tb = _T // 16
    dd = _D * _D
    grid = rows // tb
    zero = lambda i: (0, 0)
    return pl.pallas_call(
        _mlp_body,
        grid=(grid,),
        in_specs=[
            pl.BlockSpec((tb, 128), lambda i: (i, 0)),
            pl.BlockSpec((tb, 128), lambda i: (i, 0)),
            pl.BlockSpec((_D, dd), zero),
            pl.BlockSpec((1, dd), zero),
            pl.BlockSpec((dd, dd), zero),
            pl.BlockSpec((1, dd), zero),
            pl.BlockSpec((_D, dd), zero),
            pl.BlockSpec((dd, _D), zero),
        ],
        out_specs=pl.BlockSpec((tb, 128), lambda i: (i, 0)),
        out_shape=jax.ShapeDtypeStruct((rows, 128), jnp.float32),
    )(ea, xs, W1, b1, W2, b2, rep, ssum)


def _combine_body(sums, cnt, x, R, bias, out, *, relu):
    tot = sums[0] + sums[1]
    c = cnt[0, :, 0:1] + cnt[1, :, 0:1]
    agg = tot / jnp.maximum(c, 1.0)
    o = agg + jnp.dot(x[...], R[...], preferred_element_type=jnp.float32) + bias[...]
    out[...] = jnp.maximum(o, 0.0) if relu else o


def _combine_call(sums, cnt, x, R, bias, relu):
    return pl.pallas_call(
        functools.partial(_combine_body, relu=relu),
        out_shape=jax.ShapeDtypeStruct((_N, _D), jnp.float32),
    )(sums, cnt, x, R, bias)


def kernel(x, edge_index, edge_attr, W1_0, b1_0, W2_0, b2_0, R_0, bias_0,
           W1_1, b1_1, W2_1, b2_1, R_1, bias_1):
    e = edge_attr.shape[0]
    nchunk = -(-e // (_NW * _CHUNK))
    e_pad = _NW * _CHUNK * nchunk
    pad = e_pad - e

    src = jnp.concatenate(
        [edge_index[0], jnp.zeros((pad,), jnp.int32)]).reshape(_NW, nchunk, _CHUNK)
    dst = jnp.concatenate(
        [edge_index[1], jnp.full((pad,), _N, jnp.int32)]).reshape(_NW, nchunk, _CHUNK)
    ea3 = jnp.pad(edge_attr, ((0, pad), (0, 0))).reshape(
        e_pad // _CHUNK, _CHUNK, _DE).transpose(0, 2, 1)

    zeros = jnp.zeros((_N + _NS, _D), jnp.float32)
    ones = jnp.ones((_CHUNK, _D), jnp.float32)
    rep = jnp.asarray(np.kron(np.eye(_D), np.ones((1, _D))), jnp.float32)
    ssum = jnp.asarray(np.kron(np.ones((_D, 1)), np.eye(_D)), jnp.float32)

    gather = _make_gather(e_pad, nchunk)
    scatter0 = _make_scatter(e_pad, nchunk, True)
    scatter1 = _make_scatter(e_pad, nchunk, False)
    ea = _make_eapack(e_pad, nchunk)(ea3).reshape(e_pad // 16, 128)

    layers = [
        (W1_0, b1_0, W2_0, b2_0, R_0, bias_0, True),
        (W1_1, b1_1, W2_1, b2_1, R_1, bias_1, False),
    ]
    h = x
    cnt = None
    for W1, b1, W2, b2, R, bias, relu in layers:
        xs = gather(h, src).reshape(e_pad // 16, 128)
        W1p = jnp.pad(W1, ((0, _D - _DE), (0, 0)))
        msg = _mlp_call(ea, xs, W1p, b1.reshape(1, -1), W2, b2.reshape(1, -1),
                        rep, ssum)
        msg = msg.reshape(e_pad // _CHUNK, _CHUNK, _D)
        if cnt is None:
            sums, cnt = scatter0(msg, dst, zeros, ones)
        else:
            sums = scatter1(msg, dst, zeros)
        h = _combine_call(sums, cnt, h, R, bias.reshape(1, -1), relu)
    return h


# speedup vs baseline: 2.1832x; 1.2795x over previous
import functools

import numpy as np
import jax
import jax.numpy as jnp
from jax import lax
from jax.experimental import pallas as pl
from jax.experimental.pallas import tpu as pltpu
from jax.experimental.pallas import tpu_sc as plsc

_N = 10000
_D = 8
_DE = 4
_NC = 2
_NS = 16
_NW = _NC * _NS
_CHUNK = 128
_LAG = 8
_TB = 512


def _sc_mesh():
    return plsc.VectorSubcoreMesh(core_axis_name="c", subcore_axis_name="s")


def _make_gather(e_pad, nchunk):
    epw = nchunk * _CHUNK

    @functools.partial(
        pl.kernel,
        mesh=_sc_mesh(),
        out_type=jax.ShapeDtypeStruct((e_pad // _CHUNK, _CHUNK, _D),
                                      jnp.float32),
        scratch_types=[
            pltpu.VMEM((nchunk, _CHUNK), jnp.int32),
            pltpu.VMEM((nchunk, _CHUNK, _D), jnp.float32),
            pltpu.SemaphoreType.DMA,
        ],
        compiler_params=pltpu.CompilerParams(use_tc_tiling_on_sc=False),
    )
    def gather(table_hbm, idx_hbm, out_hbm, idx_v, rows_v, sem):
        wid = lax.axis_index("s") * _NC + lax.axis_index("c")
        pltpu.sync_copy(idx_hbm.at[wid], idx_v)

        def fire(j):
            pltpu.async_copy(
                table_hbm.at[idx_v.at[j]],
                rows_v.at[j],
                sem,
            )

        for j in range(_LAG):
            fire(j)

        def step(j, carry):
            @pl.when(j + _LAG < nchunk)
            def _():
                fire(j + _LAG)

            pltpu.make_async_copy(
                table_hbm.at[idx_v.at[j]],
                rows_v.at[j],
                sem,
            ).wait()
            return carry

        lax.fori_loop(0, nchunk, step, 0)
        pltpu.sync_copy(rows_v, out_hbm.at[pl.ds(wid * nchunk, nchunk)])

    return gather


def _make_eapack(e_pad, nchunk):
    nblk = e_pad // _CHUNK

    @functools.partial(
        pl.kernel,
        mesh=_sc_mesh(),
        out_type=jax.ShapeDtypeStruct((nblk, _CHUNK * _D), jnp.float32),
        scratch_types=[
            pltpu.VMEM((nchunk, _DE, _CHUNK), jnp.float32),
            pltpu.VMEM((nchunk, _CHUNK * _D), jnp.float32),
        ],
        compiler_params=pltpu.CompilerParams(use_tc_tiling_on_sc=False,
                                             needs_layout_passes=False),
    )
    def eapack(ea3_hbm, out_hbm, strip_v, rows_v):
        wid = lax.axis_index("s") * _NC + lax.axis_index("c")
        pltpu.sync_copy(ea3_hbm.at[pl.ds(wid * nchunk, nchunk)], strip_v)
        zeros16 = jnp.zeros((16,), jnp.float32)
        lane = lax.iota(jnp.int32, 16) * _D

        def step(j, carry):
            jv = jnp.zeros((16,), jnp.int32) + j
            for g in range(_CHUNK // 16):
                pos0 = lane + (g * 16 * _D)
                for f in range(_D):
                    if f < _DE:
                        val = strip_v[j, f, pl.ds(g * 16, 16)]
                    else:
                        val = zeros16
                    plsc.store_scatter(rows_v, [jv, pos0 + f], val)
            return carry

        lax.fori_loop(0, nchunk, step, 0)
        pltpu.sync_copy(rows_v, out_hbm.at[pl.ds(wid * nchunk, nchunk)])

    return eapack


def _make_scatter(e_pad, nchunk, with_counts):
    epw = nchunk * _CHUNK
    nacc = _N + _NS
    stripe_i = nacc // _NS
    stripe_o = _N // _NS

    out_type = [jax.ShapeDtypeStruct((_NC, _N, _D), jnp.float32)]
    scratch = [
        pltpu.VMEM((nchunk, _CHUNK), jnp.int32),
        pltpu.VMEM((nchunk, _CHUNK, _D), jnp.float32),
        pltpu.VMEM_SHARED((nacc, _D), jnp.float32),
    ]
    if with_counts:
        out_type.append(jax.ShapeDtypeStruct((_NC, _N, _D), jnp.float32))
        scratch.append(pltpu.VMEM((_CHUNK, _D), jnp.float32))
        scratch.append(pltpu.VMEM_SHARED((nacc, _D), jnp.float32))
    scratch.append(pltpu.SemaphoreType.DMA)
    if with_counts:
        scratch.append(pltpu.SemaphoreType.DMA)

    def body(*refs):
        if with_counts:
            (msg_hbm, idx_hbm, zeros_hbm, ones_hbm, sums_hbm, cnt_hbm,
             idx_v, msg_v, acc_sh, ones_v, cnt_sh, sem, sem2) = refs
        else:
            (msg_hbm, idx_hbm, zeros_hbm, sums_hbm,
             idx_v, msg_v, acc_sh, sem) = refs
        c = lax.axis_index("c")
        s = lax.axis_index("s")
        wid = s * _NC + c
        pltpu.sync_copy(idx_hbm.at[wid], idx_v)
        pltpu.sync_copy(msg_hbm.at[pl.ds(wid * nchunk, nchunk)], msg_v)
        pltpu.sync_copy(
            zeros_hbm.at[pl.ds(s * stripe_i, stripe_i)],
            acc_sh.at[pl.ds(s * stripe_i, stripe_i)],
        )
        if with_counts:
            pltpu.sync_copy(ones_hbm, ones_v)
            pltpu.sync_copy(
                zeros_hbm.at[pl.ds(s * stripe_i, stripe_i)],
                cnt_sh.at[pl.ds(s * stripe_i, stripe_i)],
            )
        plsc.subcore_barrier()

        def fire(j):
            pltpu.async_copy(
                msg_v.at[j],
                acc_sh.at[idx_v.at[j]],
                sem,
                add=True,
            )
            if with_counts:
                pltpu.async_copy(ones_v, cnt_sh.at[idx_v.at[j]], sem2,
                                 add=True)

        for j in range(_LAG):
            fire(j)

        def step(j, carry):
            @pl.when(j + _LAG < nchunk)
            def _():
                fire(j + _LAG)

            pltpu.make_async_copy(
                msg_v.at[j],
                acc_sh.at[idx_v.at[j]],
                sem,
            ).wait()
            if with_counts:
                pltpu.make_async_copy(
                    ones_v, cnt_sh.at[idx_v.at[j]], sem2).wait()
            return carry

        lax.fori_loop(0, nchunk, step, 0)
        plsc.subcore_barrier()
        pltpu.sync_copy(
            acc_sh.at[pl.ds(s * stripe_o, stripe_o)],
            sums_hbm.at[c, pl.ds(s * stripe_o, stripe_o)],
        )
        if with_counts:
            pltpu.sync_copy(
                cnt_sh.at[pl.ds(s * stripe_o, stripe_o)],
                cnt_hbm.at[c, pl.ds(s * stripe_o, stripe_o)],
            )

    return functools.partial(
        pl.kernel,
        mesh=_sc_mesh(),
        out_type=tuple(out_type) if with_counts else out_type[0],
        scratch_types=scratch,
        compiler_params=pltpu.CompilerParams(use_tc_tiling_on_sc=False),
    )(body)


def _mlp_body(ea, xs, W1, b1, W2, b2, rep, ssum, out):
    eap = ea[...]
    xsp = xs[...]
    W1v = W1[...]
    b1v = b1[...]
    W2v = W2[...]
    b2v = b2[...]
    repv = rep[...]
    ssumv = ssum[...]
    W2b = W2v.astype(jnp.bfloat16)
    repb = repv.astype(jnp.bfloat16)
    pieces = []
    for k in range(16):
        ea_k = eap[:, k * _D:(k + 1) * _D]
        xs_k = xsp[:, k * _D:(k + 1) * _D]
        h_k = jnp.maximum(
            jnp.dot(ea_k, W1v, preferred_element_type=jnp.float32) + b1v, 0.0)
        w_k = (jnp.dot(h_k.astype(jnp.bfloat16), W2b,
                       preferred_element_type=jnp.float32)
               + b2v).astype(jnp.bfloat16)
        xr_k = jnp.dot(xs_k.astype(jnp.bfloat16), repb,
                       preferred_element_type=jnp.float32).astype(jnp.bfloat16)
        pieces.append(
            jnp.dot(xr_k * w_k, ssumv.astype(jnp.bfloat16),
                    preferred_element_type=jnp.float32))
    out[...] = jnp.concatenate(pieces, axis=1)


def _mlp_call(ea, xs, W1, b1, W2, b2, rep, ssum):
    rows = ea.shape[0]
    tb = _TB
    dd = _D * _D
    grid = rows // tb
    zero = lambda i: (0, 0)
    return pl.pallas_call(
        _mlp_body,
        grid=(grid,),
        in_specs=[
            pl.BlockSpec((tb, 128), lambda i: (i, 0)),
            pl.BlockSpec((tb, 128), lambda i: (i, 0)),
            pl.BlockSpec((_D, dd), zero),
            pl.BlockSpec((1, dd), zero),
            pl.BlockSpec((dd, dd), zero),
            pl.BlockSpec((1, dd), zero),
            pl.BlockSpec((_D, dd), zero),
            pl.BlockSpec((dd, _D), zero),
        ],
        out_specs=pl.BlockSpec((tb, 128), lambda i: (i, 0)),
        out_shape=jax.ShapeDtypeStruct((rows, 128), jnp.float32),
    )(ea, xs, W1, b1, W2, b2, rep, ssum)


def _combine_body(sums, cnt, x, R, bias, out, *, relu):
    tot = sums[0] + sums[1]
    c = cnt[0, :, 0:1] + cnt[1, :, 0:1]
    agg = tot / jnp.maximum(c, 1.0)
    o = agg + jnp.dot(x[...], R[...], preferred_element_type=jnp.float32) + bias[...]
    out[...] = jnp.maximum(o, 0.0) if relu else o


def _combine_call(sums, cnt, x, R, bias, relu):
    return pl.pallas_call(
        functools.partial(_combine_body, relu=relu),
        out_shape=jax.ShapeDtypeStruct((_N, _D), jnp.float32),
    )(sums, cnt, x, R, bias)


def kernel(x, edge_index, edge_attr, W1_0, b1_0, W2_0, b2_0, R_0, bias_0,
           W1_1, b1_1, W2_1, b2_1, R_1, bias_1):
    e = edge_attr.shape[0]
    nchunk = -(-e // (_NW * _CHUNK))
    nchunk += nchunk % 2
    e_pad = _NW * _CHUNK * nchunk
    pad = e_pad - e

    src = jnp.concatenate(
        [edge_index[0], jnp.zeros((pad,), jnp.int32)]).reshape(_NW, nchunk, _CHUNK)
    dst = jnp.concatenate(
        [edge_index[1], jnp.full((pad,), _N, jnp.int32)]).reshape(_NW, nchunk, _CHUNK)
    ea3 = jnp.pad(edge_attr, ((0, pad), (0, 0))).reshape(
        e_pad // _CHUNK, _CHUNK, _DE).transpose(0, 2, 1)

    zeros = jnp.zeros((_N + _NS, _D), jnp.float32)
    ones = jnp.ones((_CHUNK, _D), jnp.float32)
    rep = jnp.asarray(np.kron(np.eye(_D), np.ones((1, _D))), jnp.float32)
    ssum = jnp.asarray(np.kron(np.ones((_D, 1)), np.eye(_D)), jnp.float32)

    gather = _make_gather(e_pad, nchunk)
    scatter0 = _make_scatter(e_pad, nchunk, True)
    scatter1 = _make_scatter(e_pad, nchunk, False)
    ea = _make_eapack(e_pad, nchunk)(ea3).reshape(e_pad // 16, 128)

    layers = [
        (W1_0, b1_0, W2_0, b2_0, R_0, bias_0, True),
        (W1_1, b1_1, W2_1, b2_1, R_1, bias_1, False),
    ]
    h = x
    cnt = None
    for W1, b1, W2, b2, R, bias, relu in layers:
        xs = gather(h, src).reshape(e_pad // 16, 128)
        W1p = jnp.pad(W1, ((0, _D - _DE), (0, 0)))
        msg = _mlp_call(ea, xs, W1p, b1.reshape(1, -1), W2, b2.reshape(1, -1),
                        rep, ssum)
        msg = msg.reshape(e_pad // _CHUNK, _CHUNK, _D)
        if cnt is None:
            sums, cnt = scatter0(msg, dst, zeros, ones)
        else:
            sums = scatter1(msg, dst, zeros)
        h = _combine_call(sums, cnt, h, R, bias.reshape(1, -1), relu)
    return h


# speedup vs baseline: 2.4271x; 1.1117x over previous
import functools

import numpy as np
import jax
import jax.numpy as jnp
from jax import lax
from jax.experimental import pallas as pl
from jax.experimental.pallas import tpu as pltpu
from jax.experimental.pallas import tpu_sc as plsc

_N = 10000
_D = 8
_DE = 4
_NC = 2
_NS = 16
_NW = _NC * _NS
_CHUNK = 128
_LAG = 8
_TB = 1024


def _sc_mesh():
    return plsc.VectorSubcoreMesh(core_axis_name="c", subcore_axis_name="s")


def _make_gather(e_pad, nchunk):
    epw = nchunk * _CHUNK

    @functools.partial(
        pl.kernel,
        mesh=_sc_mesh(),
        out_type=jax.ShapeDtypeStruct((e_pad // _CHUNK, _CHUNK, _D),
                                      jnp.float32),
        scratch_types=[
            pltpu.VMEM((nchunk, _CHUNK), jnp.int32),
            pltpu.VMEM((nchunk, _CHUNK, _D), jnp.float32),
            pltpu.SemaphoreType.DMA,
        ],
        compiler_params=pltpu.CompilerParams(use_tc_tiling_on_sc=False),
    )
    def gather(table_hbm, idx_hbm, out_hbm, idx_v, rows_v, sem):
        wid = lax.axis_index("s") * _NC + lax.axis_index("c")
        pltpu.sync_copy(idx_hbm.at[wid], idx_v)

        def fire(j):
            pltpu.async_copy(
                table_hbm.at[idx_v.at[j]],
                rows_v.at[j],
                sem,
            )

        for j in range(_LAG):
            fire(j)

        def step(j, carry):
            @pl.when(j + _LAG < nchunk)
            def _():
                fire(j + _LAG)

            pltpu.make_async_copy(
                table_hbm.at[idx_v.at[j]],
                rows_v.at[j],
                sem,
            ).wait()
            return carry

        lax.fori_loop(0, nchunk, step, 0)
        pltpu.sync_copy(rows_v, out_hbm.at[pl.ds(wid * nchunk, nchunk)])

    return gather


def _make_eapack(e_pad, nchunk):
    nblk = e_pad // _CHUNK

    @functools.partial(
        pl.kernel,
        mesh=_sc_mesh(),
        out_type=jax.ShapeDtypeStruct((nblk, _CHUNK * _D), jnp.float32),
        scratch_types=[
            pltpu.VMEM((nchunk, _DE, _CHUNK), jnp.float32),
            pltpu.VMEM((nchunk, _CHUNK * _D), jnp.float32),
        ],
        compiler_params=pltpu.CompilerParams(use_tc_tiling_on_sc=False,
                                             needs_layout_passes=False),
    )
    def eapack(ea3_hbm, out_hbm, strip_v, rows_v):
        wid = lax.axis_index("s") * _NC + lax.axis_index("c")
        pltpu.sync_copy(ea3_hbm.at[pl.ds(wid * nchunk, nchunk)], strip_v)
        zeros16 = jnp.zeros((16,), jnp.float32)
        lane = lax.iota(jnp.int32, 16) * _D

        def step(j, carry):
            jv = jnp.zeros((16,), jnp.int32) + j
            for g in range(_CHUNK // 16):
                pos0 = lane + (g * 16 * _D)
                for f in range(_D):
                    if f < _DE:
                        val = strip_v[j, f, pl.ds(g * 16, 16)]
                    else:
                        val = zeros16
                    plsc.store_scatter(rows_v, [jv, pos0 + f], val)
            return carry

        lax.fori_loop(0, nchunk, step, 0)
        pltpu.sync_copy(rows_v, out_hbm.at[pl.ds(wid * nchunk, nchunk)])

    return eapack


def _make_scatter(e_pad, nchunk, with_counts):
    epw = nchunk * _CHUNK
    nacc = _N + _NS
    stripe_i = nacc // _NS
    stripe_o = _N // _NS

    out_type = [jax.ShapeDtypeStruct((_NC, _N, _D), jnp.float32)]
    scratch = [
        pltpu.VMEM((nchunk, _CHUNK), jnp.int32),
        pltpu.VMEM((nchunk, _CHUNK, _D), jnp.float32),
        pltpu.VMEM_SHARED((nacc, _D), jnp.float32),
    ]
    if with_counts:
        out_type.append(jax.ShapeDtypeStruct((_NC, _N, _D), jnp.float32))
        scratch.append(pltpu.VMEM((_CHUNK, _D), jnp.float32))
        scratch.append(pltpu.VMEM_SHARED((nacc, _D), jnp.float32))
    scratch.append(pltpu.SemaphoreType.DMA)
    if with_counts:
        scratch.append(pltpu.SemaphoreType.DMA)

    def body(*refs):
        if with_counts:
            (msg_hbm, idx_hbm, zeros_hbm, ones_hbm, sums_hbm, cnt_hbm,
             idx_v, msg_v, acc_sh, ones_v, cnt_sh, sem, sem2) = refs
        else:
            (msg_hbm, idx_hbm, zeros_hbm, sums_hbm,
             idx_v, msg_v, acc_sh, sem) = refs
        c = lax.axis_index("c")
        s = lax.axis_index("s")
        wid = s * _NC + c
        pltpu.sync_copy(idx_hbm.at[wid], idx_v)
        pltpu.sync_copy(msg_hbm.at[pl.ds(wid * nchunk, nchunk)], msg_v)
        pltpu.sync_copy(
            zeros_hbm.at[pl.ds(s * stripe_i, stripe_i)],
            acc_sh.at[pl.ds(s * stripe_i, stripe_i)],
        )
        if with_counts:
            pltpu.sync_copy(ones_hbm, ones_v)
            pltpu.sync_copy(
                zeros_hbm.at[pl.ds(s * stripe_i, stripe_i)],
                cnt_sh.at[pl.ds(s * stripe_i, stripe_i)],
            )
        plsc.subcore_barrier()

        def fire(j):
            pltpu.async_copy(
                msg_v.at[j],
                acc_sh.at[idx_v.at[j]],
                sem,
                add=True,
            )
            if with_counts:
                pltpu.async_copy(ones_v, cnt_sh.at[idx_v.at[j]], sem2,
                                 add=True)

        for j in range(_LAG):
            fire(j)

        def step(j, carry):
            @pl.when(j + _LAG < nchunk)
            def _():
                fire(j + _LAG)

            pltpu.make_async_copy(
                msg_v.at[j],
                acc_sh.at[idx_v.at[j]],
                sem,
            ).wait()
            if with_counts:
                pltpu.make_async_copy(
                    ones_v, cnt_sh.at[idx_v.at[j]], sem2).wait()
            return carry

        lax.fori_loop(0, nchunk, step, 0)
        plsc.subcore_barrier()
        pltpu.sync_copy(
            acc_sh.at[pl.ds(s * stripe_o, stripe_o)],
            sums_hbm.at[c, pl.ds(s * stripe_o, stripe_o)],
        )
        if with_counts:
            pltpu.sync_copy(
                cnt_sh.at[pl.ds(s * stripe_o, stripe_o)],
                cnt_hbm.at[c, pl.ds(s * stripe_o, stripe_o)],
            )

    return functools.partial(
        pl.kernel,
        mesh=_sc_mesh(),
        out_type=tuple(out_type) if with_counts else out_type[0],
        scratch_types=scratch,
        compiler_params=pltpu.CompilerParams(use_tc_tiling_on_sc=False),
    )(body)


def _mlp_body(ea, xs, W1, b1, W2, b2, rep, ssum, out):
    eap = ea[...]
    xsp = xs[...]
    W1v = W1[...]
    b1v = b1[...]
    W2v = W2[...]
    b2v = b2[...]
    repv = rep[...]
    ssumv = ssum[...]
    W2b = W2v.astype(jnp.bfloat16)
    repb = repv.astype(jnp.bfloat16)
    pieces = []
    for k in range(16):
        ea_k = eap[:, k * _D:(k + 1) * _D]
        xs_k = xsp[:, k * _D:(k + 1) * _D]
        h_k = jnp.maximum(
            jnp.dot(ea_k, W1v, preferred_element_type=jnp.float32) + b1v, 0.0)
        w_k = (jnp.dot(h_k.astype(jnp.bfloat16), W2b,
                       preferred_element_type=jnp.float32)
               + b2v).astype(jnp.bfloat16)
        xr_k = jnp.dot(xs_k.astype(jnp.bfloat16), repb,
                       preferred_element_type=jnp.float32).astype(jnp.bfloat16)
        pieces.append(
            jnp.dot(xr_k * w_k, ssumv.astype(jnp.bfloat16),
                    preferred_element_type=jnp.float32))
    out[...] = jnp.concatenate(pieces, axis=1)


def _mlp_call(ea, xs, W1, b1, W2, b2, rep, ssum):
    rows = ea.shape[0]
    tb = _TB
    dd = _D * _D
    grid = rows // tb
    zero = lambda i: (0, 0)
    return pl.pallas_call(
        _mlp_body,
        grid=(grid,),
        in_specs=[
            pl.BlockSpec((tb, 128), lambda i: (i, 0)),
            pl.BlockSpec((tb, 128), lambda i: (i, 0)),
            pl.BlockSpec((_D, dd), zero),
            pl.BlockSpec((1, dd), zero),
            pl.BlockSpec((dd, dd), zero),
            pl.BlockSpec((1, dd), zero),
            pl.BlockSpec((_D, dd), zero),
            pl.BlockSpec((dd, _D), zero),
        ],
        out_specs=pl.BlockSpec((tb, 128), lambda i: (i, 0)),
        out_shape=jax.ShapeDtypeStruct((rows, 128), jnp.float32),
    )(ea, xs, W1, b1, W2, b2, rep, ssum)


def _combine_body(sums, cnt, x, R, bias, out, *, relu):
    tot = sums[0] + sums[1]
    c = cnt[0, :, 0:1] + cnt[1, :, 0:1]
    agg = tot / jnp.maximum(c, 1.0)
    o = agg + jnp.dot(x[...], R[...], preferred_element_type=jnp.float32) + bias[...]
    out[...] = jnp.maximum(o, 0.0) if relu else o


def _combine_call(sums, cnt, x, R, bias, relu):
    return pl.pallas_call(
        functools.partial(_combine_body, relu=relu),
        out_shape=jax.ShapeDtypeStruct((_N, _D), jnp.float32),
    )(sums, cnt, x, R, bias)


def kernel(x, edge_index, edge_attr, W1_0, b1_0, W2_0, b2_0, R_0, bias_0,
           W1_1, b1_1, W2_1, b2_1, R_1, bias_1):
    e = edge_attr.shape[0]
    nchunk = -(-e // (_NW * _CHUNK))
    nchunk += nchunk % 2
    e_pad = _NW * _CHUNK * nchunk
    pad = e_pad - e

    src = jnp.concatenate(
        [edge_index[0], jnp.zeros((pad,), jnp.int32)]).reshape(_NW, nchunk, _CHUNK)
    dst = jnp.concatenate(
        [edge_index[1], jnp.full((pad,), _N, jnp.int32)]).reshape(_NW, nchunk, _CHUNK)
    ea3 = jnp.pad(edge_attr, ((0, pad), (0, 0))).reshape(
        e_pad // _CHUNK, _CHUNK, _DE).transpose(0, 2, 1)

    zeros = jnp.zeros((_N + _NS, _D), jnp.float32)
    ones = jnp.ones((_CHUNK, _D), jnp.float32)
    rep = jnp.asarray(np.kron(np.eye(_D), np.ones((1, _D))), jnp.float32)
    ssum = jnp.asarray(np.kron(np.ones((_D, 1)), np.eye(_D)), jnp.float32)

    gather = _make_gather(e_pad, nchunk)
    scatter0 = _make_scatter(e_pad, nchunk, True)
    scatter1 = _make_scatter(e_pad, nchunk, False)
    ea = _make_eapack(e_pad, nchunk)(ea3).reshape(e_pad // 16, 128)

    layers = [
        (W1_0, b1_0, W2_0, b2_0, R_0, bias_0, True),
        (W1_1, b1_1, W2_1, b2_1, R_1, bias_1, False),
    ]
    h = x
    cnt = None
    for W1, b1, W2, b2, R, bias, relu in layers:
        xs = gather(h, src).reshape(e_pad // 16, 128)
        W1p = jnp.pad(W1, ((0, _D - _DE), (0, 0)))
        msg = _mlp_call(ea, xs, W1p, b1.reshape(1, -1), W2, b2.reshape(1, -1),
                        rep, ssum)
        msg = msg.reshape(e_pad // _CHUNK, _CHUNK, _D)
        if cnt is None:
            sums, cnt = scatter0(msg, dst, zeros, ones)
        else:
            sums = scatter1(msg, dst, zeros)
        h = _combine_call(sums, cnt, h, R, bias.reshape(1, -1), relu)
    return h
